# baseline (speedup 1.0000x reference)
import functools

import jax
import jax.numpy as jnp
from jax import lax
from jax.experimental import pallas as pl
from jax.experimental.pallas import tpu as pltpu

ROI_PER_IMAGE = 512
FG_RATIO = 0.5
REG_FG_THRESH = 0.55
CLS_FG_THRESH = 0.75
CLS_BG_THRESH = 0.25

B, N, M, MP = 16, 4096, 100, 128
NFG = int(ROI_PER_IMAGE * FG_RATIO)
NBG = ROI_PER_IMAGE - NFG


def _iou_kernel(rois_ref, lab_ref, gtt_ref, gtl_ref, maxov_ref, asn_ref, bfs_ref):
    roi = rois_ref[0]
    inter = jnp.ones((N, MP), jnp.float32)
    va = jnp.ones((N, 1), jnp.float32)
    vb = jnp.ones((1, MP), jnp.float32)
    for d in range(3):
        c_r = roi[:, d:d + 1]
        s_r = roi[:, d + 3:d + 4]
        c_g = gtt_ref[0, d:d + 1, :]
        s_g = gtt_ref[0, d + 3:d + 4, :]
        amin = c_r - s_r / 2.0
        amax = c_r + s_r / 2.0
        bmin = c_g - s_g / 2.0
        bmax = c_g + s_g / 2.0
        lt = jnp.maximum(amin, bmin)
        rb = jnp.minimum(amax, bmax)
        inter = inter * jnp.maximum(rb - lt, 0.0)
        va = va * jnp.maximum(amax - amin, 0.0)
        vb = vb * jnp.maximum(bmax - bmin, 0.0)
    iou = inter / jnp.maximum(va + vb - inter, 1e-6)
    gt_valid = jnp.sum(jnp.abs(gtl_ref[0]), axis=1, keepdims=True) > 0.0
    lab_f = lab_ref[0].astype(jnp.float32)
    gt_lab = gtt_ref[0, 7:8, :]
    keep = (lab_f == gt_lab) & jnp.transpose(gt_valid)
    iou = jnp.where(keep, iou, -1.0)
    mx = jnp.max(iou, axis=1, keepdims=True)
    lane = lax.broadcasted_iota(jnp.int32, (N, MP), 1)
    asn = jnp.min(jnp.where(iou == mx, lane, MP), axis=1, keepdims=True)
    mxc = jnp.maximum(mx, 0.0)
    maxov_ref[0] = mxc
    asn_ref[0] = asn
    bfs = [jnp.where(lab_f == float(c + 1), mxc, -1.0) for c in range(3)]
    bfs_ref[0] = jnp.concatenate(bfs, axis=1)


def _sort_kernel(v_ref, smp_ref, riou_ref, reg_ref, cls_ref, itv_ref):
    v = v_ref[:]
    ix = lax.broadcasted_iota(jnp.int32, (B, N), 1)
    pos = lax.broadcasted_iota(jnp.int32, (B, N), 1)

    def roll(x, s):
        return jnp.concatenate([x[:, s:], x[:, :s]], axis=1)

    k = 2
    while k <= N:
        dirm = (pos & k) == 0
        s = k // 2
        while s >= 1:
            lowm = (pos & s) == 0
            pv = jnp.where(lowm, roll(v, s), roll(v, N - s))
            pix = jnp.where(lowm, roll(ix, s), roll(ix, N - s))
            first = (v > pv) | ((v == pv) & (ix < pix))
            keep = first == (lowm == dirm)
            v = jnp.where(keep, v, pv)
            ix = jnp.where(keep, ix, pix)
            s //= 2
        k *= 2
    smp = jnp.concatenate([ix[:, :NFG], ix[:, N - NBG:]], axis=1)
    riou = jnp.concatenate([v[:, :NFG], v[:, N - NBG:]], axis=1)
    smp_ref[:] = smp
    riou_ref[:] = riou
    reg_ref[:] = (riou > REG_FG_THRESH).astype(jnp.int32)
    fg = riou > CLS_FG_THRESH
    bg = riou < CLS_BG_THRESH
    itv = (~fg) & (~bg)
    soft = (riou - CLS_BG_THRESH) / (CLS_FG_THRESH - CLS_BG_THRESH)
    cls_ref[:] = jnp.where(itv, soft, fg.astype(jnp.float32))
    itv_ref[:] = itv.astype(jnp.int32)


def _gather_kernel(rois_ref, sc_ref, lab_ref, asn_ref, mov_ref, gtl_ref,
                   smp_ref, rois_s_ref, gtof_ref, sc_s_ref, lab_s_ref,
                   afs_ref):
    R = ROI_PER_IMAGE
    smp = smp_ref[0]
    pt = (lax.broadcasted_iota(jnp.int32, (N, R), 0) ==
          jnp.broadcast_to(smp, (N, R))).astype(jnp.float32)
    feat = jnp.concatenate([
        rois_ref[0],
        sc_ref[0],
        lab_ref[0].astype(jnp.float32),
        asn_ref[0].astype(jnp.float32),
        mov_ref[0],
    ], axis=1)
    g = lax.dot_general(pt, feat, (((0,), (0,)), ((), ())),
                        preferred_element_type=jnp.float32)
    rois_s_ref[0] = g[:, 0:7]
    sc_s_ref[0] = g[:, 7:8]
    lab_s = g[:, 8:9]
    lab_s_ref[0] = lab_s.astype(jnp.int32)
    asn_s = g[:, 9:10]
    riou = g[:, 10:11]
    q = (asn_s.astype(jnp.int32) == lax.broadcasted_iota(jnp.int32, (R, MP), 1)
         ).astype(jnp.float32)
    gtof = lax.dot_general(q, gtl_ref[0], (((1,), (0,)), ((), ())),
                           preferred_element_type=jnp.float32)
    gtof_ref[0] = gtof
    afs = [jnp.where(lab_s == float(c + 1), riou, -1.0) for c in range(3)]
    afs_ref[0] = jnp.concatenate(afs, axis=1)


def kernel(rois, roi_scores, roi_labels, gt_boxes, unlabeled_inds):
    f32, i32 = jnp.float32, jnp.int32
    gt_pad = jnp.pad(gt_boxes, ((0, 0), (0, MP - M), (0, 0)))
    gt_t = jnp.transpose(gt_pad, (0, 2, 1))
    sc_r = roi_scores.reshape(B, N, 1)
    lab_r = roi_labels.reshape(B, N, 1)

    maxov, asn, bfs = pl.pallas_call(
        _iou_kernel,
        grid=(B,),
        in_specs=[
            pl.BlockSpec((1, N, 7), lambda b: (b, 0, 0)),
            pl.BlockSpec((1, N, 1), lambda b: (b, 0, 0)),
            pl.BlockSpec((1, 8, MP), lambda b: (b, 0, 0)),
            pl.BlockSpec((1, MP, 8), lambda b: (b, 0, 0)),
        ],
        out_specs=[
            pl.BlockSpec((1, N, 1), lambda b: (b, 0, 0)),
            pl.BlockSpec((1, N, 1), lambda b: (b, 0, 0)),
            pl.BlockSpec((1, N, 3), lambda b: (b, 0, 0)),
        ],
        out_shape=[
            jax.ShapeDtypeStruct((B, N, 1), f32),
            jax.ShapeDtypeStruct((B, N, 1), i32),
            jax.ShapeDtypeStruct((B, N, 3), f32),
        ],
    )(rois, lab_r, gt_t, gt_pad)

    mov2 = maxov.reshape(B, N)
    R = ROI_PER_IMAGE
    smp, riou, reg, cls, itv = pl.pallas_call(
        _sort_kernel,
        out_shape=[
            jax.ShapeDtypeStruct((B, R), i32),
            jax.ShapeDtypeStruct((B, R), f32),
            jax.ShapeDtypeStruct((B, R), i32),
            jax.ShapeDtypeStruct((B, R), f32),
            jax.ShapeDtypeStruct((B, R), i32),
        ],
    )(mov2)

    smp_r = smp.reshape(B, 1, R)
    rois_s, gtof, sc_s, lab_s, afs = pl.pallas_call(
        _gather_kernel,
        grid=(B,),
        in_specs=[
            pl.BlockSpec((1, N, 7), lambda b: (b, 0, 0)),
            pl.BlockSpec((1, N, 1), lambda b: (b, 0, 0)),
            pl.BlockSpec((1, N, 1), lambda b: (b, 0, 0)),
            pl.BlockSpec((1, N, 1), lambda b: (b, 0, 0)),
            pl.BlockSpec((1, N, 1), lambda b: (b, 0, 0)),
            pl.BlockSpec((1, MP, 8), lambda b: (b, 0, 0)),
            pl.BlockSpec((1, 1, R), lambda b: (b, 0, 0)),
        ],
        out_specs=[
            pl.BlockSpec((1, R, 7), lambda b: (b, 0, 0)),
            pl.BlockSpec((1, R, 8), lambda b: (b, 0, 0)),
            pl.BlockSpec((1, R, 1), lambda b: (b, 0, 0)),
            pl.BlockSpec((1, R, 1), lambda b: (b, 0, 0)),
            pl.BlockSpec((1, R, 3), lambda b: (b, 0, 0)),
        ],
        out_shape=[
            jax.ShapeDtypeStruct((B, R, 7), f32),
            jax.ShapeDtypeStruct((B, R, 8), f32),
            jax.ShapeDtypeStruct((B, R, 1), f32),
            jax.ShapeDtypeStruct((B, R, 1), i32),
            jax.ShapeDtypeStruct((B, R, 3), f32),
        ],
    )(rois, sc_r, lab_r, asn, maxov, gt_pad, smp_r)

    unlab = jnp.zeros((B,), bool).at[unlabeled_inds].set(True)
    return (rois_s, gtof, riou, sc_s.reshape(B, R), lab_s.reshape(B, R),
            reg, cls, itv.astype(bool), jnp.transpose(bfs, (0, 2, 1)),
            jnp.transpose(afs, (0, 2, 1)), unlab)


# speedup vs baseline: 1.3832x; 1.3832x over previous
import jax
import jax.numpy as jnp
from jax import lax
from jax.experimental import pallas as pl
from jax.experimental.pallas import tpu as pltpu

ROI_PER_IMAGE = 512
FG_RATIO = 0.5
REG_FG_THRESH = 0.55
CLS_FG_THRESH = 0.75
CLS_BG_THRESH = 0.25

B, N, M, MP = 16, 4096, 100, 128
R = ROI_PER_IMAGE
NFG = int(R * FG_RATIO)
NBG = R - NFG


def _iou_kernel(roist_ref, lab_ref, gt_ref, mov_ref, gtof_tab_ref, bfs_ref):
    roit = roist_ref[0]
    gt = gt_ref[0]
    inter = jnp.ones((MP, N), jnp.float32)
    va = jnp.ones((1, N), jnp.float32)
    vb = jnp.ones((MP, 1), jnp.float32)
    for d in range(3):
        c_r = roit[d:d + 1, :]
        s_r = roit[d + 3:d + 4, :]
        c_g = gt[:, d:d + 1]
        s_g = gt[:, d + 3:d + 4]
        amin = c_r - s_r / 2.0
        amax = c_r + s_r / 2.0
        bmin = c_g - s_g / 2.0
        bmax = c_g + s_g / 2.0
        inter = inter * jnp.maximum(jnp.minimum(amax, bmax) -
                                    jnp.maximum(amin, bmin), 0.0)
        va = va * jnp.maximum(amax - amin, 0.0)
        vb = vb * jnp.maximum(bmax - bmin, 0.0)
    iou = inter / jnp.maximum(va + vb - inter, 1e-6)
    gt_valid = jnp.sum(jnp.abs(gt), axis=1, keepdims=True) > 0.0
    lab_f = lab_ref[0].astype(jnp.float32)
    keep = (lab_f == gt[:, 7:8]) & gt_valid
    iou = jnp.where(keep, iou, -1.0)
    mx = jnp.max(iou, axis=0, keepdims=True)
    sub = lax.broadcasted_iota(jnp.int32, (MP, N), 0)
    asn = jnp.min(jnp.where(iou == mx, sub, MP), axis=0, keepdims=True)
    mov_ref[0] = jnp.maximum(mx, 0.0)
    oh = (sub == asn).astype(jnp.float32)
    gtof_tab_ref[0] = lax.dot_general(oh, gt, (((0,), (0,)), ((), ())),
                                      preferred_element_type=jnp.float32)
    mxc = jnp.maximum(mx, 0.0)
    bfs = [jnp.where(lab_f == float(c + 1), mxc, -1.0) for c in range(3)]
    bfs_ref[0] = jnp.concatenate(bfs, axis=0)


def _sample_kernel(mov_ref, rois_ref, sc_ref, lab_ref, gtof_tab_ref,
                   riou_o, reg_o, cls_o, itv_o, rois_s_ref, gtof_s_ref,
                   sc_s_ref, lab_s_ref, afs_ref, smp_s, riou_s):
    b = pl.program_id(0)

    @pl.when(b == 0)
    def _sort():
        v = mov_ref[:, 0, :]
        ix = lax.broadcasted_iota(jnp.int32, (B, N), 1)
        pos = lax.broadcasted_iota(jnp.int32, (B, N), 1)

        def roll(x, s):
            return jnp.concatenate([x[:, s:], x[:, :s]], axis=1)

        k = 2
        while k <= N:
            dirm = (pos & k) == 0
            s = k // 2
            while s >= 1:
                lowm = (pos & s) == 0
                pv = jnp.where(lowm, roll(v, s), roll(v, N - s))
                pix = jnp.where(lowm, roll(ix, s), roll(ix, N - s))
                first = (v > pv) | ((v == pv) & (ix < pix))
                keepm = first == (lowm == dirm)
                v = jnp.where(keepm, v, pv)
                ix = jnp.where(keepm, ix, pix)
                s //= 2
            k *= 2
        smp_s[:] = jnp.concatenate([ix[:, :NFG], ix[:, N - NBG:]], axis=1)
        riou = jnp.concatenate([v[:, :NFG], v[:, N - NBG:]], axis=1)
        riou_s[:] = riou
        riou_o[:] = riou
        reg_o[:] = (riou > REG_FG_THRESH).astype(jnp.int32)
        fg = riou > CLS_FG_THRESH
        bg = riou < CLS_BG_THRESH
        itv = (~fg) & (~bg)
        soft = (riou - CLS_BG_THRESH) / (CLS_FG_THRESH - CLS_BG_THRESH)
        cls_o[:] = jnp.where(itv, soft, fg.astype(jnp.float32))
        itv_o[:] = itv.astype(jnp.int32)

    smp = smp_s[pl.ds(b, 1), :]
    pt = (lax.broadcasted_iota(jnp.int32, (N, R), 0) ==
          jnp.broadcast_to(smp, (N, R))).astype(jnp.float32)
    feat = jnp.concatenate([
        rois_ref[0],
        sc_ref[0],
        lab_ref[0].astype(jnp.float32),
        gtof_tab_ref[0],
    ], axis=1)
    g = lax.dot_general(pt, feat, (((0,), (0,)), ((), ())),
                        preferred_element_type=jnp.float32)
    rois_s_ref[0] = g[:, 0:7]
    sc_s_ref[0] = g[:, 7:8]
    lab_col = g[:, 8:9]
    lab_s_ref[0] = lab_col.astype(jnp.int32)
    gtof_s_ref[0] = g[:, 9:17]
    lab_row = jnp.transpose(lab_col)
    riou_row = riou_s[pl.ds(b, 1), :]
    afs = [jnp.where(lab_row == float(c + 1), riou_row, -1.0)
           for c in range(3)]
    afs_ref[0] = jnp.concatenate(afs, axis=0)


def kernel(rois, roi_scores, roi_labels, gt_boxes, unlabeled_inds):
    f32, i32 = jnp.float32, jnp.int32
    gt_pad = jnp.pad(gt_boxes, ((0, 0), (0, MP - M), (0, 0)))
    rois_t = jnp.transpose(rois, (0, 2, 1))
    lab_row = roi_labels.reshape(B, 1, N)
    sc_col = roi_scores.reshape(B, N, 1)
    lab_col = roi_labels.reshape(B, N, 1)

    mov, gtof_tab, bfs = pl.pallas_call(
        _iou_kernel,
        grid=(B,),
        in_specs=[
            pl.BlockSpec((1, 7, N), lambda b: (b, 0, 0)),
            pl.BlockSpec((1, 1, N), lambda b: (b, 0, 0)),
            pl.BlockSpec((1, MP, 8), lambda b: (b, 0, 0)),
        ],
        out_specs=[
            pl.BlockSpec((1, 1, N), lambda b: (b, 0, 0)),
            pl.BlockSpec((1, N, 8), lambda b: (b, 0, 0)),
            pl.BlockSpec((1, 3, N), lambda b: (b, 0, 0)),
        ],
        out_shape=[
            jax.ShapeDtypeStruct((B, 1, N), f32),
            jax.ShapeDtypeStruct((B, N, 8), f32),
            jax.ShapeDtypeStruct((B, 3, N), f32),
        ],
    )(rois_t, lab_row, gt_pad)

    (riou, reg, cls, itv, rois_s, gtof_s, sc_s, lab_s, afs) = pl.pallas_call(
        _sample_kernel,
        grid=(B,),
        in_specs=[
            pl.BlockSpec((B, 1, N), lambda b: (0, 0, 0)),
            pl.BlockSpec((1, N, 7), lambda b: (b, 0, 0)),
            pl.BlockSpec((1, N, 1), lambda b: (b, 0, 0)),
            pl.BlockSpec((1, N, 1), lambda b: (b, 0, 0)),
            pl.BlockSpec((1, N, 8), lambda b: (b, 0, 0)),
        ],
        out_specs=[
            pl.BlockSpec((B, R), lambda b: (0, 0)),
            pl.BlockSpec((B, R), lambda b: (0, 0)),
            pl.BlockSpec((B, R), lambda b: (0, 0)),
            pl.BlockSpec((B, R), lambda b: (0, 0)),
            pl.BlockSpec((1, R, 7), lambda b: (b, 0, 0)),
            pl.BlockSpec((1, R, 8), lambda b: (b, 0, 0)),
            pl.BlockSpec((1, R, 1), lambda b: (b, 0, 0)),
            pl.BlockSpec((1, R, 1), lambda b: (b, 0, 0)),
            pl.BlockSpec((1, 3, R), lambda b: (b, 0, 0)),
        ],
        out_shape=[
            jax.ShapeDtypeStruct((B, R), f32),
            jax.ShapeDtypeStruct((B, R), i32),
            jax.ShapeDtypeStruct((B, R), f32),
            jax.ShapeDtypeStruct((B, R), i32),
            jax.ShapeDtypeStruct((B, R, 7), f32),
            jax.ShapeDtypeStruct((B, R, 8), f32),
            jax.ShapeDtypeStruct((B, R, 1), f32),
            jax.ShapeDtypeStruct((B, R, 1), i32),
            jax.ShapeDtypeStruct((B, 3, R), f32),
        ],
        scratch_shapes=[
            pltpu.VMEM((B, R), i32),
            pltpu.VMEM((B, R), f32),
        ],
    )(mov, rois, sc_col, lab_col, gtof_tab)

    unlab = jnp.zeros((B,), bool).at[unlabeled_inds].set(True)
    return (rois_s, gtof_s, riou, sc_s.reshape(B, R), lab_s.reshape(B, R),
            reg, cls, itv.astype(bool), bfs, afs, unlab)


# speedup vs baseline: 1.5900x; 1.1495x over previous
import jax
import jax.numpy as jnp
from jax import lax
from jax.experimental import pallas as pl
from jax.experimental.pallas import tpu as pltpu

ROI_PER_IMAGE = 512
FG_RATIO = 0.5
REG_FG_THRESH = 0.55
CLS_FG_THRESH = 0.75
CLS_BG_THRESH = 0.25

B, N, M, MP = 16, 4096, 100, 128
R = ROI_PER_IMAGE
NFG = int(R * FG_RATIO)
NBG = R - NFG


def _body(roist_ref, labrow_ref, gt_ref, rois_ref, sc_ref, labcol_ref,
          riou_o, reg_o, cls_o, itv_o, bfs_ref, rois_s_ref, gtof_s_ref,
          sc_s_ref, lab_s_ref, afs_ref,
          mov_s, gtof_tab_s, smp_s, riou_s):
    b = pl.program_id(0)

    @pl.when(b < B)
    def _iou():
        roit = roist_ref[0]
        gt = gt_ref[0]
        inter = jnp.ones((MP, N), jnp.float32)
        va = jnp.ones((1, N), jnp.float32)
        vb = jnp.ones((MP, 1), jnp.float32)
        for d in range(3):
            c_r = roit[d:d + 1, :]
            s_r = roit[d + 3:d + 4, :]
            c_g = gt[:, d:d + 1]
            s_g = gt[:, d + 3:d + 4]
            amin = c_r - s_r / 2.0
            amax = c_r + s_r / 2.0
            bmin = c_g - s_g / 2.0
            bmax = c_g + s_g / 2.0
            inter = inter * jnp.maximum(jnp.minimum(amax, bmax) -
                                        jnp.maximum(amin, bmin), 0.0)
            va = va * jnp.maximum(amax - amin, 0.0)
            vb = vb * jnp.maximum(bmax - bmin, 0.0)
        iou = inter / jnp.maximum(va + vb - inter, 1e-6)
        gt_valid = jnp.sum(jnp.abs(gt), axis=1, keepdims=True) > 0.0
        lab_f = labrow_ref[0].astype(jnp.float32)
        keep = (lab_f == gt[:, 7:8]) & gt_valid
        iou = jnp.where(keep, iou, -1.0)
        mx = jnp.max(iou, axis=0, keepdims=True)
        sub = lax.broadcasted_iota(jnp.int32, (MP, N), 0)
        asn = jnp.min(jnp.where(iou == mx, sub, MP), axis=0, keepdims=True)
        mxc = jnp.maximum(mx, 0.0)
        mov_s[pl.ds(b, 1), :] = mxc
        oh = (sub == asn).astype(jnp.float32)
        tab = lax.dot_general(oh, gt, (((0,), (0,)), ((), ())),
                              preferred_element_type=jnp.float32)
        gtof_tab_s[pl.ds(b, 1), :, :] = tab.reshape(1, N, 8)
        bfs = [jnp.where(lab_f == float(c + 1), mxc, -1.0) for c in range(3)]
        bfs_ref[0] = jnp.concatenate(bfs, axis=0)

    @pl.when(b == B - 1)
    def _sort():
        v = mov_s[:, :]
        ix = lax.broadcasted_iota(jnp.int32, (B, N), 1)
        pos = lax.broadcasted_iota(jnp.int32, (B, N), 1)

        def roll(x, s):
            return jnp.concatenate([x[:, s:], x[:, :s]], axis=1)

        k = 2
        while k <= N:
            dirm = (pos & k) == 0
            s = k // 2
            while s >= 1:
                lowm = (pos & s) == 0
                pv = jnp.where(lowm, roll(v, s), roll(v, N - s))
                pix = jnp.where(lowm, roll(ix, s), roll(ix, N - s))
                first = (v > pv) | ((v == pv) & (ix < pix))
                keepm = first == (lowm == dirm)
                v = jnp.where(keepm, v, pv)
                ix = jnp.where(keepm, ix, pix)
                s //= 2
            k *= 2
        smp_s[:] = jnp.concatenate([ix[:, :NFG], ix[:, N - NBG:]], axis=1)
        riou = jnp.concatenate([v[:, :NFG], v[:, N - NBG:]], axis=1)
        riou_s[:] = riou
        riou_o[:] = riou
        reg_o[:] = (riou > REG_FG_THRESH).astype(jnp.int32)
        fg = riou > CLS_FG_THRESH
        bg = riou < CLS_BG_THRESH
        itv = (~fg) & (~bg)
        soft = (riou - CLS_BG_THRESH) / (CLS_FG_THRESH - CLS_BG_THRESH)
        cls_o[:] = jnp.where(itv, soft, fg.astype(jnp.float32))
        itv_o[:] = itv.astype(jnp.int32)

    @pl.when(b >= B)
    def _gather():
        i = b - B
        smp = smp_s[pl.ds(i, 1), :]
        pt = (lax.broadcasted_iota(jnp.int32, (N, R), 0) ==
              jnp.broadcast_to(smp, (N, R))).astype(jnp.float32)
        feat = jnp.concatenate([
            rois_ref[0],
            sc_ref[0],
            labcol_ref[0].astype(jnp.float32),
            gtof_tab_s[pl.ds(i, 1), :, :][0],
        ], axis=1)
        g = lax.dot_general(pt, feat, (((0,), (0,)), ((), ())),
                            preferred_element_type=jnp.float32)
        rois_s_ref[0] = g[:, 0:7]
        sc_s_ref[0] = g[:, 7:8]
        lab_col = g[:, 8:9]
        lab_s_ref[0] = lab_col.astype(jnp.int32)
        gtof_s_ref[0] = g[:, 9:17]
        lab_row = jnp.transpose(lab_col)
        riou_row = riou_s[pl.ds(i, 1), :]
        afs = [jnp.where(lab_row == float(c + 1), riou_row, -1.0)
               for c in range(3)]
        afs_ref[0] = jnp.concatenate(afs, axis=0)


def kernel(rois, roi_scores, roi_labels, gt_boxes, unlabeled_inds):
    f32, i32 = jnp.float32, jnp.int32
    gt_pad = jnp.pad(gt_boxes, ((0, 0), (0, MP - M), (0, 0)))
    rois_t = jnp.transpose(rois, (0, 2, 1))
    lab_row = roi_labels.reshape(B, 1, N)
    sc_col = roi_scores.reshape(B, N, 1)
    lab_col = roi_labels.reshape(B, N, 1)

    lo = lambda b: (jnp.minimum(b, B - 1), 0, 0)
    hi = lambda b: (jnp.maximum(b - B, 0), 0, 0)
    zz = lambda b: (0, 0)

    (riou, reg, cls, itv, bfs, rois_s, gtof_s, sc_s, lab_s,
     afs) = pl.pallas_call(
        _body,
        grid=(2 * B,),
        in_specs=[
            pl.BlockSpec((1, 7, N), lo),
            pl.BlockSpec((1, 1, N), lo),
            pl.BlockSpec((1, MP, 8), lo),
            pl.BlockSpec((1, N, 7), hi),
            pl.BlockSpec((1, N, 1), hi),
            pl.BlockSpec((1, N, 1), hi),
        ],
        out_specs=[
            pl.BlockSpec((B, R), zz),
            pl.BlockSpec((B, R), zz),
            pl.BlockSpec((B, R), zz),
            pl.BlockSpec((B, R), zz),
            pl.BlockSpec((1, 3, N), lo),
            pl.BlockSpec((1, R, 7), hi),
            pl.BlockSpec((1, R, 8), hi),
            pl.BlockSpec((1, R, 1), hi),
            pl.BlockSpec((1, R, 1), hi),
            pl.BlockSpec((1, 3, R), hi),
        ],
        out_shape=[
            jax.ShapeDtypeStruct((B, R), f32),
            jax.ShapeDtypeStruct((B, R), i32),
            jax.ShapeDtypeStruct((B, R), f32),
            jax.ShapeDtypeStruct((B, R), i32),
            jax.ShapeDtypeStruct((B, 3, N), f32),
            jax.ShapeDtypeStruct((B, R, 7), f32),
            jax.ShapeDtypeStruct((B, R, 8), f32),
            jax.ShapeDtypeStruct((B, R, 1), f32),
            jax.ShapeDtypeStruct((B, R, 1), i32),
            jax.ShapeDtypeStruct((B, 3, R), f32),
        ],
        scratch_shapes=[
            pltpu.VMEM((B, N), f32),
            pltpu.VMEM((B, N, 8), f32),
            pltpu.VMEM((B, R), i32),
            pltpu.VMEM((B, R), f32),
        ],
    )(rois_t, lab_row, gt_pad, rois, sc_col, lab_col)

    unlab = jnp.zeros((B,), bool).at[unlabeled_inds].set(True)
    return (rois_s, gtof_s, riou, sc_s.reshape(B, R), lab_s.reshape(B, R),
            reg, cls, itv.astype(bool), bfs, afs, unlab)
